# Initial kernel scaffold; baseline (speedup 1.0000x reference)
import jax
import jax.numpy as jnp
from jax.experimental import pallas as pl


def kernel(x, edge_index, W1l, b1, W1r, W2l, b2, W2r):
    raise NotImplementedError("write your pallas kernel here")


# speedup vs baseline: 4.8490x; 4.8490x over previous
import functools

import jax
import jax.numpy as jnp
from jax import lax
from jax.experimental import pallas as pl
from jax.experimental.pallas import tpu as pltpu
from jax.experimental.pallas import tpu_sc as plsc

NC = 2
NS = 16
NW = NC * NS

CH = 125
KJ = 8
HH = 64


def _tc_proj(x, Wl, Wr, b):
    N, D = x.shape
    H = Wl.shape[1]
    R = 200
    G = N // R

    def body(x_ref, wl_ref, wr_ref, b_ref, p0_ref, p1_ref, s_ref):
        xb = x_ref[...]
        p = jnp.dot(xb, wl_ref[...], preferred_element_type=jnp.float32)
        p0_ref[...] = p[:, :HH]
        p1_ref[...] = p[:, HH:]
        s_ref[...] = jnp.dot(xb, wr_ref[...], preferred_element_type=jnp.float32) + b_ref[...]

    return pl.pallas_call(
        body,
        grid=(G,),
        in_specs=[
            pl.BlockSpec((R, D), lambda i: (i, 0)),
            pl.BlockSpec((D, H), lambda i: (0, 0)),
            pl.BlockSpec((D, H), lambda i: (0, 0)),
            pl.BlockSpec((1, H), lambda i: (0, 0)),
        ],
        out_specs=[
            pl.BlockSpec((R, HH), lambda i: (i, 0)),
            pl.BlockSpec((R, HH), lambda i: (i, 0)),
            pl.BlockSpec((R, H), lambda i: (i, 0)),
        ],
        out_shape=[
            jax.ShapeDtypeStruct((N, HH), jnp.float32),
            jax.ShapeDtypeStruct((N, HH), jnp.float32),
            jax.ShapeDtypeStruct((N, H), jnp.float32),
        ],
    )(x, Wl, Wr, b.reshape(1, H))


def _tc_combine(acc, deg, S):
    N, H = S.shape
    R = 200
    G = N // R

    def body(a00, a01, a10, a11, d0, d1, s_ref, o_ref):
        d = d0[:, 0:1] + d1[:, 0:1]
        a = jnp.concatenate((a00[...] + a10[...], a01[...] + a11[...]), axis=1)
        o_ref[...] = jnp.maximum(a / jnp.maximum(d, 1.0) + s_ref[...], 0.0)

    half = pl.BlockSpec((R, HH), lambda i: (i, 0))
    degs = pl.BlockSpec((R, 16), lambda i: (i, 0))
    full = pl.BlockSpec((R, H), lambda i: (i, 0))
    return pl.pallas_call(
        body,
        grid=(G,),
        in_specs=[half, half, half, half, degs, degs, full],
        out_specs=full,
        out_shape=jax.ShapeDtypeStruct((N, H), jnp.float32),
    )(acc[0, 0], acc[0, 1], acc[1, 0], acc[1, 1], deg[0], deg[1], S)


def _sc_segment(P0, P1, src3, dst3):
    N = P0.shape[0]
    KB = src3.shape[1] // KJ
    RCH = 200
    NCH = N // RCH
    CPT = -(-NCH // NS)
    HL = HH // 16

    mesh = plsc.VectorSubcoreMesh(
        core_axis_name="c", subcore_axis_name="s", num_cores=NC, num_subcores=NS
    )

    @functools.partial(
        pl.kernel,
        out_type=[
            jax.ShapeDtypeStruct((NC, 2, N, HH), jnp.float32),
            jax.ShapeDtypeStruct((NC, N, 16), jnp.float32),
        ],
        mesh=mesh,
        scratch_types=[
            pltpu.VMEM((KJ, CH), jnp.int32),
            pltpu.VMEM((KJ, CH), jnp.int32),
            pltpu.VMEM((CH, HH), jnp.float32),
            pltpu.VMEM((CH, 16), jnp.float32),
            pltpu.VMEM((RCH, HH), jnp.float32),
            pltpu.VMEM((RCH, 16), jnp.float32),
            pltpu.VMEM_SHARED((N, HH), jnp.float32),
            pltpu.VMEM_SHARED((N, 16), jnp.float32),
            pltpu.SemaphoreType.DMA,
        ],
        compiler_params=pltpu.CompilerParams(use_tc_tiling_on_sc=False),
    )
    def seg(p0_hbm, p1_hbm, src_hbm, dst_hbm, acc_out, deg_out,
            srcv, dstv, rows, small, cbuf, csmall, acc_sh, deg_sh, sem):
        c = lax.axis_index("c")
        s = lax.axis_index("s")
        w = s * NC + c
        z16 = jnp.zeros((16,), jnp.float32)
        one16 = jnp.ones((16,), jnp.float32)

        def zbuf_body(i, carry):
            cbuf[i // HL, pl.ds((i % HL) * 16, 16)] = z16
            return carry

        lax.fori_loop(0, RCH * HL, zbuf_body, 0)

        def zsmall_body(i, carry):
            csmall[i, :] = z16
            return carry

        lax.fori_loop(0, RCH, zsmall_body, 0)

        def ones_body(i, carry):
            small[i, :] = one16
            return carry

        lax.fori_loop(0, CH, ones_body, 0)

        for hc in range(2):
            p_hbm = (p0_hbm, p1_hbm)[hc]
            for i in range(CPT):
                t = s + NS * i
                @pl.when(t < NCH)
                def _():
                    pltpu.sync_copy(cbuf, acc_sh.at[pl.ds(t * RCH, RCH)])
                    if hc == 0:
                        pltpu.sync_copy(csmall, deg_sh.at[pl.ds(t * RCH, RCH)])
            plsc.subcore_barrier()

            def blk(kb, carry):
                pltpu.sync_copy(src_hbm.at[w, pl.ds(kb * KJ, KJ)], srcv)
                pltpu.sync_copy(dst_hbm.at[w, pl.ds(kb * KJ, KJ)], dstv)
                for j in range(KJ):
                    pltpu.async_copy(p_hbm.at[srcv.at[j]], rows, sem).wait()
                    pltpu.sync_copy(rows, acc_sh.at[dstv.at[j]], add=True)
                    if hc == 0:
                        pltpu.sync_copy(small, deg_sh.at[dstv.at[j]], add=True)
                return carry

            lax.fori_loop(0, KB, blk, 0)
            plsc.subcore_barrier()

            for i in range(CPT):
                t = s + NS * i
                @pl.when(t < NCH)
                def _():
                    r0 = t * RCH
                    pltpu.sync_copy(acc_sh.at[pl.ds(r0, RCH)], cbuf)
                    pltpu.sync_copy(cbuf, acc_out.at[c, hc, pl.ds(r0, RCH)])
                    if hc == 0:
                        pltpu.sync_copy(deg_sh.at[pl.ds(r0, RCH)], csmall)
                        pltpu.sync_copy(csmall, deg_out.at[c, pl.ds(r0, RCH)])
            if hc == 0:
                lax.fori_loop(0, RCH * HL, zbuf_body, 0)

    return seg(P0, P1, src3, dst3)


def kernel(x, edge_index, W1l, b1, W1r, W2l, b2, W2r):
    E = edge_index.shape[1]
    src3 = edge_index[0].reshape(NW, E // (NW * CH), CH)
    dst3 = edge_index[1].reshape(NW, E // (NW * CH), CH)

    P0, P1, S1 = _tc_proj(x, W1l, W1r, b1)
    acc1, deg = _sc_segment(P0, P1, src3, dst3)
    h = _tc_combine(acc1, deg, S1)

    Q0, Q1, S2 = _tc_proj(h, W2l, W2r, b2)
    acc2, _ = _sc_segment(Q0, Q1, src3, dst3)
    return _tc_combine(acc2, deg, S2)


# speedup vs baseline: 6.2471x; 1.2883x over previous
import functools

import jax
import jax.numpy as jnp
from jax import lax
from jax.experimental import pallas as pl
from jax.experimental.pallas import tpu as pltpu
from jax.experimental.pallas import tpu_sc as plsc

NC = 2
NS = 16
NW = NC * NS

CH = 125
HH = 64
NB = 4


def _tc_proj(x, Wl, Wr, b):
    N, D = x.shape
    H = Wl.shape[1]
    R = 200
    G = N // R

    def body(x_ref, wl_ref, wr_ref, b_ref, p0_ref, p1_ref, s_ref):
        xb = x_ref[...]
        p = jnp.dot(xb, wl_ref[...], preferred_element_type=jnp.float32)
        p0_ref[...] = p[:, :HH]
        p1_ref[...] = p[:, HH:]
        s_ref[...] = jnp.dot(xb, wr_ref[...], preferred_element_type=jnp.float32) + b_ref[...]

    return pl.pallas_call(
        body,
        grid=(G,),
        in_specs=[
            pl.BlockSpec((R, D), lambda i: (i, 0)),
            pl.BlockSpec((D, H), lambda i: (0, 0)),
            pl.BlockSpec((D, H), lambda i: (0, 0)),
            pl.BlockSpec((1, H), lambda i: (0, 0)),
        ],
        out_specs=[
            pl.BlockSpec((R, HH), lambda i: (i, 0)),
            pl.BlockSpec((R, HH), lambda i: (i, 0)),
            pl.BlockSpec((R, H), lambda i: (i, 0)),
        ],
        out_shape=[
            jax.ShapeDtypeStruct((N, HH), jnp.float32),
            jax.ShapeDtypeStruct((N, HH), jnp.float32),
            jax.ShapeDtypeStruct((N, H), jnp.float32),
        ],
    )(x, Wl, Wr, b.reshape(1, H))


def _tc_combine(acc, deg, S):
    N, H = S.shape
    R = 200
    G = N // R

    def body(a00, a01, a10, a11, d0, d1, s_ref, o_ref):
        d = d0[:, 0:1] + d1[:, 0:1]
        a = jnp.concatenate((a00[...] + a10[...], a01[...] + a11[...]), axis=1)
        o_ref[...] = jnp.maximum(a / jnp.maximum(d, 1.0) + s_ref[...], 0.0)

    half = pl.BlockSpec((R, HH), lambda i: (i, 0))
    degs = pl.BlockSpec((R, 16), lambda i: (i, 0))
    full = pl.BlockSpec((R, H), lambda i: (i, 0))
    return pl.pallas_call(
        body,
        grid=(G,),
        in_specs=[half, half, half, half, degs, degs, full],
        out_specs=full,
        out_shape=jax.ShapeDtypeStruct((N, H), jnp.float32),
    )(acc[0, 0], acc[0, 1], acc[1, 0], acc[1, 1], deg[0], deg[1], S)


def _sc_segment(P0, P1, src3, dst3):
    N = P0.shape[0]
    NCHK = src3.shape[1]
    NR = NCHK // NB
    RCH = 200
    NCH = N // RCH
    CPT = -(-NCH // NS)
    HL = HH // 16

    mesh = plsc.VectorSubcoreMesh(
        core_axis_name="c", subcore_axis_name="s", num_cores=NC, num_subcores=NS
    )

    @functools.partial(
        pl.kernel,
        out_type=jax.ShapeDtypeStruct((NC, 2, N, HH), jnp.float32),
        mesh=mesh,
        scratch_types=[
            pltpu.VMEM((NCHK, CH), jnp.int32),
            pltpu.VMEM((NCHK, CH), jnp.int32),
            pltpu.VMEM((NB, CH, HH), jnp.float32),
            pltpu.VMEM((RCH, HH), jnp.float32),
            pltpu.VMEM((RCH, HH), jnp.float32),
            pltpu.VMEM_SHARED((N, HH), jnp.float32),
            pltpu.SemaphoreType.DMA((NB,)),
            pltpu.SemaphoreType.DMA((NB,)),
        ],
        compiler_params=pltpu.CompilerParams(use_tc_tiling_on_sc=False),
    )
    def seg(p0_hbm, p1_hbm, src_hbm, dst_hbm, acc_out,
            srcv, dstv, ring, zbuf, obuf, acc_sh, gsem, ssem):
        c = lax.axis_index("c")
        s = lax.axis_index("s")
        w = s * NC + c
        z16 = jnp.zeros((16,), jnp.float32)

        pltpu.sync_copy(src_hbm.at[w], srcv)
        pltpu.sync_copy(dst_hbm.at[w], dstv)

        def zbuf_body(i, carry):
            zbuf[i // HL, pl.ds((i % HL) * 16, 16)] = z16
            return carry

        lax.fori_loop(0, RCH * HL, zbuf_body, 0)

        for hc in range(2):
            p_hbm = (p0_hbm, p1_hbm)[hc]
            for i in range(CPT):
                t = s + NS * i
                @pl.when(t < NCH)
                def _():
                    pltpu.sync_copy(zbuf, acc_sh.at[pl.ds(t * RCH, RCH)])
            plsc.subcore_barrier()

            pltpu.async_copy(p_hbm.at[srcv.at[0]], ring.at[0], gsem.at[0])

            def rnd(r, carry):
                for b in range(NB):
                    k = r * NB + b
                    pltpu.make_async_copy(
                        p_hbm.at[srcv.at[k]], ring.at[b], gsem.at[b]).wait()
                    pltpu.async_copy(
                        ring.at[b], acc_sh.at[dstv.at[k]], ssem.at[b], add=True)
                    if b < NB - 1:
                        kn = k + 1
                        @pl.when(r >= 1)
                        def _():
                            pltpu.make_async_copy(
                                ring.at[b + 1], acc_sh.at[dstv.at[kn]],
                                ssem.at[b + 1]).wait()
                        pltpu.async_copy(
                            p_hbm.at[srcv.at[kn]], ring.at[b + 1], gsem.at[b + 1])
                    else:
                        kn = k + 1
                        pltpu.make_async_copy(
                            ring.at[0], acc_sh.at[dstv.at[0]], ssem.at[0]).wait()
                        @pl.when(r < NR - 1)
                        def _():
                            pltpu.async_copy(
                                p_hbm.at[srcv.at[kn]], ring.at[0], gsem.at[0])
                return carry

            lax.fori_loop(0, NR, rnd, 0)

            for b in range(1, NB):
                pltpu.make_async_copy(
                    ring.at[b], acc_sh.at[dstv.at[b]], ssem.at[b]).wait()
            plsc.subcore_barrier()

            for i in range(CPT):
                t = s + NS * i
                @pl.when(t < NCH)
                def _():
                    r0 = t * RCH
                    pltpu.sync_copy(acc_sh.at[pl.ds(r0, RCH)], obuf)
                    pltpu.sync_copy(obuf, acc_out.at[c, hc, pl.ds(r0, RCH)])

    return seg(P0, P1, src3, dst3)


def _sc_degree(dst3, N):
    NCHK = dst3.shape[1]
    RCH = 200
    NCH = N // RCH
    CPT = -(-NCH // NS)
    LAG = 8

    mesh = plsc.VectorSubcoreMesh(
        core_axis_name="c", subcore_axis_name="s", num_cores=NC, num_subcores=NS
    )

    @functools.partial(
        pl.kernel,
        out_type=jax.ShapeDtypeStruct((NC, N, 16), jnp.float32),
        mesh=mesh,
        scratch_types=[
            pltpu.VMEM((NCHK, CH), jnp.int32),
            pltpu.VMEM((CH, 16), jnp.float32),
            pltpu.VMEM((RCH, 16), jnp.float32),
            pltpu.VMEM_SHARED((N, 16), jnp.float32),
            pltpu.SemaphoreType.DMA,
        ],
        compiler_params=pltpu.CompilerParams(use_tc_tiling_on_sc=False),
    )
    def degk(dst_hbm, deg_out, dstv, ones_b, small, deg_sh, dsem):
        c = lax.axis_index("c")
        s = lax.axis_index("s")
        w = s * NC + c
        z16 = jnp.zeros((16,), jnp.float32)
        one16 = jnp.ones((16,), jnp.float32)

        pltpu.sync_copy(dst_hbm.at[w], dstv)

        def zsmall_body(i, carry):
            small[i, :] = z16
            return carry

        lax.fori_loop(0, RCH, zsmall_body, 0)

        def ones_body(i, carry):
            ones_b[i, :] = one16
            return carry

        lax.fori_loop(0, CH, ones_body, 0)

        for i in range(CPT):
            t = s + NS * i
            @pl.when(t < NCH)
            def _():
                pltpu.sync_copy(small, deg_sh.at[pl.ds(t * RCH, RCH)])
        plsc.subcore_barrier()

        def blk(k, carry):
            pltpu.async_copy(ones_b, deg_sh.at[dstv.at[k]], dsem, add=True)
            @pl.when(k >= LAG)
            def _():
                pltpu.make_async_copy(
                    ones_b, deg_sh.at[dstv.at[0]], dsem).wait()
            return carry

        lax.fori_loop(0, NCHK, blk, 0)
        for _ in range(LAG):
            pltpu.make_async_copy(ones_b, deg_sh.at[dstv.at[0]], dsem).wait()
        plsc.subcore_barrier()

        for i in range(CPT):
            t = s + NS * i
            @pl.when(t < NCH)
            def _():
                r0 = t * RCH
                pltpu.sync_copy(deg_sh.at[pl.ds(r0, RCH)], small)
                pltpu.sync_copy(small, deg_out.at[c, pl.ds(r0, RCH)])

    return degk(dst3)


def kernel(x, edge_index, W1l, b1, W1r, W2l, b2, W2r):
    E = edge_index.shape[1]
    src3 = edge_index[0].reshape(NW, E // (NW * CH), CH)
    dst3 = edge_index[1].reshape(NW, E // (NW * CH), CH)

    deg = _sc_degree(dst3, x.shape[0])
    P0, P1, S1 = _tc_proj(x, W1l, W1r, b1)
    acc1 = _sc_segment(P0, P1, src3, dst3)
    h = _tc_combine(acc1, deg, S1)

    Q0, Q1, S2 = _tc_proj(h, W2l, W2r, b2)
    acc2 = _sc_segment(Q0, Q1, src3, dst3)
    return _tc_combine(acc2, deg, S2)


# speedup vs baseline: 7.2346x; 1.1581x over previous
import functools

import jax
import jax.numpy as jnp
from jax import lax
from jax.experimental import pallas as pl
from jax.experimental.pallas import tpu as pltpu
from jax.experimental.pallas import tpu_sc as plsc

NC = 2
NS = 16
NW = NC * NS
CH = 125
HH = 64
NB = 4
RCH = 200


def _tc_proj(x, Wl, Wr, b):
    N, D = x.shape
    H = Wl.shape[1]
    R = 1000
    G = N // R

    def body(x_ref, wl_ref, wr_ref, b_ref, p0_ref, p1_ref, s_ref):
        xb = x_ref[...]
        pv = jnp.dot(xb, wl_ref[...], preferred_element_type=jnp.float32)
        p0_ref[...] = pv[:, :HH]
        p1_ref[...] = pv[:, HH:]
        s_ref[...] = jnp.dot(xb, wr_ref[...], preferred_element_type=jnp.float32) + b_ref[...]

    return pl.pallas_call(
        body,
        grid=(G,),
        in_specs=[
            pl.BlockSpec((R, D), lambda i: (i, 0)),
            pl.BlockSpec((D, H), lambda i: (0, 0)),
            pl.BlockSpec((D, H), lambda i: (0, 0)),
            pl.BlockSpec((1, H), lambda i: (0, 0)),
        ],
        out_specs=[
            pl.BlockSpec((R, HH), lambda i: (i, 0)),
            pl.BlockSpec((R, HH), lambda i: (i, 0)),
            pl.BlockSpec((R, H), lambda i: (i, 0)),
        ],
        out_shape=[
            jax.ShapeDtypeStruct((N, HH), jnp.float32),
            jax.ShapeDtypeStruct((N, HH), jnp.float32),
            jax.ShapeDtypeStruct((N, H), jnp.float32),
        ],
    )(x, Wl, Wr, b.reshape(1, H))


def _tc_combine(acc, deg, S):
    N, H = S.shape
    R = 1000
    G = N // R

    def body(a00, a01, a10, a11, d0, d1, s_ref, o_ref):
        d = d0[:, 0:1] + d1[:, 0:1]
        a = jnp.concatenate((a00[...] + a10[...], a01[...] + a11[...]), axis=1)
        o_ref[...] = jnp.maximum(a / jnp.maximum(d, 1.0) + s_ref[...], 0.0)

    half = pl.BlockSpec((R, HH), lambda i: (i, 0))
    degs = pl.BlockSpec((R, 16), lambda i: (i, 0))
    full = pl.BlockSpec((R, H), lambda i: (i, 0))
    return pl.pallas_call(
        body,
        grid=(G,),
        in_specs=[half, half, half, half, degs, degs, full],
        out_specs=full,
        out_shape=jax.ShapeDtypeStruct((N, H), jnp.float32),
    )(acc[0, 0], acc[0, 1], acc[1, 0], acc[1, 1], deg[0], deg[1], S)


def _sc_segment(P0, P1, src3, dst3):
    N = P0.shape[0]
    NCHK = dst3.shape[1]
    NR = NCHK // NB
    NA = N + 8
    NCH = N // RCH
    CPT = -(-NCH // NS)
    HL = HH // 16

    mesh = plsc.VectorSubcoreMesh(
        core_axis_name="c", subcore_axis_name="s", num_cores=NC, num_subcores=NS
    )

    @functools.partial(
        pl.kernel,
        out_type=jax.ShapeDtypeStruct((NC, 2, N, HH), jnp.float32),
        mesh=mesh,
        scratch_types=[
            pltpu.VMEM((NCHK, CH), jnp.int32),
            pltpu.VMEM((NCHK, CH), jnp.int32),
            pltpu.VMEM((NB, CH, HH), jnp.float32),
            pltpu.VMEM((RCH, HH), jnp.float32),
            pltpu.VMEM((RCH, HH), jnp.float32),
            pltpu.VMEM_SHARED((NA, HH), jnp.float32),
            pltpu.SemaphoreType.DMA((NB,)),
            pltpu.SemaphoreType.DMA((NB,)),
        ],
        compiler_params=pltpu.CompilerParams(use_tc_tiling_on_sc=False),
    )
    def seg(p0_hbm, p1_hbm, src_hbm, dst_hbm, acc_out,
            srcv, dstv, ring, zbuf, obuf, acc_sh, gsem, ssem):
        c = lax.axis_index("c")
        s = lax.axis_index("s")
        w = s * NC + c
        z16 = jnp.zeros((16,), jnp.float32)

        pltpu.sync_copy(src_hbm.at[w], srcv)
        pltpu.sync_copy(dst_hbm.at[w], dstv)

        def zbuf_body(i, carry):
            zbuf[i // HL, pl.ds((i % HL) * 16, 16)] = z16
            return carry

        lax.fori_loop(0, RCH * HL, zbuf_body, 0)

        for hc in range(2):
            p_hbm = (p0_hbm, p1_hbm)[hc]
            for i in range(CPT):
                t = s + NS * i
                @pl.when(t < NCH)
                def _():
                    pltpu.sync_copy(zbuf, acc_sh.at[pl.ds(t * RCH, RCH)])
            plsc.subcore_barrier()

            idx = srcv
            pltpu.async_copy(p_hbm.at[idx.at[0]], ring.at[0], gsem.at[0])

            def rnd(r, carry):
                for b in range(NB):
                    k = r * NB + b
                    pltpu.make_async_copy(
                        p_hbm.at[idx.at[k]], ring.at[b], gsem.at[b]).wait()
                    pltpu.async_copy(
                        ring.at[b], acc_sh.at[dstv.at[k]], ssem.at[b], add=True)
                    if b < NB - 1:
                        kn = k + 1
                        @pl.when(r >= 1)
                        def _():
                            pltpu.make_async_copy(
                                ring.at[b + 1], acc_sh.at[dstv.at[kn]],
                                ssem.at[b + 1]).wait()
                        pltpu.async_copy(
                            p_hbm.at[idx.at[kn]], ring.at[b + 1], gsem.at[b + 1])
                    else:
                        kn = k + 1
                        pltpu.make_async_copy(
                            ring.at[0], acc_sh.at[dstv.at[0]], ssem.at[0]).wait()
                        @pl.when(r < NR - 1)
                        def _():
                            pltpu.async_copy(
                                p_hbm.at[idx.at[kn]], ring.at[0], gsem.at[0])
                return carry

            lax.fori_loop(0, NR, rnd, 0)

            for b in range(1, NB):
                pltpu.make_async_copy(
                    ring.at[b], acc_sh.at[dstv.at[b]], ssem.at[b]).wait()
            plsc.subcore_barrier()

            for i in range(CPT):
                t = s + NS * i
                @pl.when(t < NCH)
                def _():
                    r0 = t * RCH
                    pltpu.sync_copy(acc_sh.at[pl.ds(r0, RCH)], obuf)
                    pltpu.sync_copy(
                        obuf, acc_out.at[c, hc, pl.ds(r0, RCH)])

    return seg(P0, P1, src3, dst3)


def _sc_degree(dst3, N):
    NCHK = dst3.shape[1]
    NA = N + 8
    NCH = N // RCH
    CPT = -(-NCH // NS)
    LAG = 8

    mesh = plsc.VectorSubcoreMesh(
        core_axis_name="c", subcore_axis_name="s", num_cores=NC, num_subcores=NS
    )

    @functools.partial(
        pl.kernel,
        out_type=jax.ShapeDtypeStruct((NC, N, 16), jnp.float32),
        mesh=mesh,
        scratch_types=[
            pltpu.VMEM((NCHK, CH), jnp.int32),
            pltpu.VMEM((CH, 16), jnp.float32),
            pltpu.VMEM((RCH, 16), jnp.float32),
            pltpu.VMEM_SHARED((NA, 16), jnp.float32),
            pltpu.SemaphoreType.DMA,
        ],
        compiler_params=pltpu.CompilerParams(use_tc_tiling_on_sc=False),
    )
    def degk(dst_hbm, deg_out, dstv, ones_b, small, deg_sh, dsem):
        c = lax.axis_index("c")
        s = lax.axis_index("s")
        w = s * NC + c
        z16 = jnp.zeros((16,), jnp.float32)
        one16 = jnp.ones((16,), jnp.float32)

        pltpu.sync_copy(dst_hbm.at[w], dstv)

        def zsmall_body(i, carry):
            small[i, :] = z16
            return carry

        lax.fori_loop(0, RCH, zsmall_body, 0)

        def ones_body(i, carry):
            ones_b[i, :] = one16
            return carry

        lax.fori_loop(0, CH, ones_body, 0)

        for i in range(CPT):
            t = s + NS * i
            @pl.when(t < NCH)
            def _():
                pltpu.sync_copy(small, deg_sh.at[pl.ds(t * RCH, RCH)])
        plsc.subcore_barrier()

        def blk(k, carry):
            pltpu.async_copy(ones_b, deg_sh.at[dstv.at[k]], dsem, add=True)
            @pl.when(k >= LAG)
            def _():
                pltpu.make_async_copy(
                    ones_b, deg_sh.at[dstv.at[0]], dsem).wait()
            return carry

        lax.fori_loop(0, NCHK, blk, 0)
        for _ in range(LAG):
            pltpu.make_async_copy(ones_b, deg_sh.at[dstv.at[0]], dsem).wait()
        plsc.subcore_barrier()

        for i in range(CPT):
            t = s + NS * i
            @pl.when(t < NCH)
            def _():
                r0 = t * RCH
                pltpu.sync_copy(deg_sh.at[pl.ds(r0, RCH)], small)
                pltpu.sync_copy(small, deg_out.at[c, pl.ds(r0, RCH)])

    return degk(dst3)


def kernel(x, edge_index, W1l, b1, W1r, W2l, b2, W2r):
    N = x.shape[0]
    E = edge_index.shape[1]
    ET = E // NW
    ETP = -(-ET // CH) * CH
    src3 = edge_index[0].reshape(NW, ET // CH, CH)
    dst3 = edge_index[1].reshape(NW, ET // CH, CH)

    deg = _sc_degree(dst3, N)
    Pa, Pb, S1 = _tc_proj(x, W1l, W1r, b1)
    acc1 = _sc_segment(Pa, Pb, src3, dst3)
    h = _tc_combine(acc1, deg, S1)

    Qa, Qb, S2 = _tc_proj(h, W2l, W2r, b2)
    acc2 = _sc_segment(Qa, Qb, src3, dst3)
    return _tc_combine(acc2, deg, S2)


# speedup vs baseline: 8.5082x; 1.1760x over previous
import functools

import jax
import jax.numpy as jnp
from jax import lax
from jax.experimental import pallas as pl
from jax.experimental.pallas import tpu as pltpu
from jax.experimental.pallas import tpu_sc as plsc

NC = 2
NS = 16
NW = NC * NS
CH = 125
HH = 64
NB = 4
RCH = 200


def _tc_proj(x, Wl, Wr, b):
    N, D = x.shape
    H = Wl.shape[1]
    R = 1000
    G = N // R

    def body(x_ref, wl_ref, wr_ref, b_ref, p0_ref, p1_ref, s_ref):
        xb = x_ref[...]
        pv = jnp.dot(xb, wl_ref[...], preferred_element_type=jnp.float32)
        p0_ref[...] = pv[:, :HH]
        p1_ref[...] = pv[:, HH:]
        s_ref[...] = jnp.dot(xb, wr_ref[...], preferred_element_type=jnp.float32) + b_ref[...]

    return pl.pallas_call(
        body,
        grid=(G,),
        in_specs=[
            pl.BlockSpec((R, D), lambda i: (i, 0)),
            pl.BlockSpec((D, H), lambda i: (0, 0)),
            pl.BlockSpec((D, H), lambda i: (0, 0)),
            pl.BlockSpec((1, H), lambda i: (0, 0)),
        ],
        out_specs=[
            pl.BlockSpec((R, HH), lambda i: (i, 0)),
            pl.BlockSpec((R, HH), lambda i: (i, 0)),
            pl.BlockSpec((R, H), lambda i: (i, 0)),
        ],
        out_shape=[
            jax.ShapeDtypeStruct((N, HH), jnp.float32),
            jax.ShapeDtypeStruct((N, HH), jnp.float32),
            jax.ShapeDtypeStruct((N, H), jnp.float32),
        ],
    )(x, Wl, Wr, b.reshape(1, H))


def _tc_combine(acc, deg, S):
    N, H = S.shape
    R = 1000
    G = N // R

    def body(a0_ref, a1_ref, d0_ref, d1_ref, s_ref, o_ref):
        d = d0_ref[0, :, 0:1] + d1_ref[0, :, 0:1]
        a = a0_ref[0] + a1_ref[0]
        o_ref[...] = jnp.maximum(a / jnp.maximum(d, 1.0) + s_ref[...], 0.0)

    full = pl.BlockSpec((R, H), lambda i: (i, 0))
    c0 = pl.BlockSpec((1, R, 128), lambda i: (0, i, 0))
    c1 = pl.BlockSpec((1, R, 128), lambda i: (1, i, 0))
    return pl.pallas_call(
        body,
        grid=(G,),
        in_specs=[c0, c1, c0, c1, full],
        out_specs=full,
        out_shape=jax.ShapeDtypeStruct((N, H), jnp.float32),
    )(acc, acc, deg, deg, S)


def _sc_segment(P0, P1, src3, dst3):
    N = P0.shape[0]
    NCHK = dst3.shape[1]
    NR = NCHK // NB
    NA = N + 8
    NCH = N // RCH
    CPT = -(-NCH // NS)
    HL = HH // 16

    mesh = plsc.VectorSubcoreMesh(
        core_axis_name="c", subcore_axis_name="s", num_cores=NC, num_subcores=NS
    )

    @functools.partial(
        pl.kernel,
        out_type=jax.ShapeDtypeStruct((NC, N, 2 * HH), jnp.float32),
        mesh=mesh,
        scratch_types=[
            pltpu.VMEM((NCHK, CH), jnp.int32),
            pltpu.VMEM((NCHK, CH), jnp.int32),
            pltpu.VMEM((NB, CH, HH), jnp.float32),
            pltpu.VMEM((RCH, HH), jnp.float32),
            pltpu.VMEM((RCH, HH), jnp.float32),
            pltpu.VMEM_SHARED((NA, HH), jnp.float32),
            pltpu.SemaphoreType.DMA((NB,)),
            pltpu.SemaphoreType.DMA((NB,)),
        ],
        compiler_params=pltpu.CompilerParams(use_tc_tiling_on_sc=False),
    )
    def seg(p0_hbm, p1_hbm, src_hbm, dst_hbm, acc_out,
            srcv, dstv, ring, zbuf, obuf, acc_sh, gsem, ssem):
        c = lax.axis_index("c")
        s = lax.axis_index("s")
        w = s * NC + c
        z16 = jnp.zeros((16,), jnp.float32)

        pltpu.sync_copy(src_hbm.at[w], srcv)
        pltpu.sync_copy(dst_hbm.at[w], dstv)

        def zbuf_body(i, carry):
            zbuf[i // HL, pl.ds((i % HL) * 16, 16)] = z16
            return carry

        lax.fori_loop(0, RCH * HL, zbuf_body, 0)

        for hc in range(2):
            p_hbm = (p0_hbm, p1_hbm)[hc]
            for i in range(CPT):
                t = s + NS * i
                @pl.when(t < NCH)
                def _():
                    pltpu.sync_copy(zbuf, acc_sh.at[pl.ds(t * RCH, RCH)])
            plsc.subcore_barrier()

            idx = srcv
            pltpu.async_copy(p_hbm.at[idx.at[0]], ring.at[0], gsem.at[0])

            def rnd(r, carry):
                for b in range(NB):
                    k = r * NB + b
                    pltpu.make_async_copy(
                        p_hbm.at[idx.at[k]], ring.at[b], gsem.at[b]).wait()
                    pltpu.async_copy(
                        ring.at[b], acc_sh.at[dstv.at[k]], ssem.at[b], add=True)
                    if b < NB - 1:
                        kn = k + 1
                        @pl.when(r >= 1)
                        def _():
                            pltpu.make_async_copy(
                                ring.at[b + 1], acc_sh.at[dstv.at[kn]],
                                ssem.at[b + 1]).wait()
                        pltpu.async_copy(
                            p_hbm.at[idx.at[kn]], ring.at[b + 1], gsem.at[b + 1])
                    else:
                        kn = k + 1
                        pltpu.make_async_copy(
                            ring.at[0], acc_sh.at[dstv.at[0]], ssem.at[0]).wait()
                        @pl.when(r < NR - 1)
                        def _():
                            pltpu.async_copy(
                                p_hbm.at[idx.at[kn]], ring.at[0], gsem.at[0])
                return carry

            lax.fori_loop(0, NR, rnd, 0)

            for b in range(1, NB):
                pltpu.make_async_copy(
                    ring.at[b], acc_sh.at[dstv.at[b]], ssem.at[b]).wait()
            plsc.subcore_barrier()

            for i in range(CPT):
                t = s + NS * i
                @pl.when(t < NCH)
                def _():
                    r0 = t * RCH
                    pltpu.sync_copy(acc_sh.at[pl.ds(r0, RCH)], obuf)
                    pltpu.sync_copy(
                        obuf, acc_out.at[c, pl.ds(r0, RCH), pl.ds(hc * HH, HH)])

    return seg(P0, P1, src3, dst3)


def _sc_degree(dst3, N):
    NCHK = dst3.shape[1]
    NA = N + 8
    NCH = N // RCH
    CPT = -(-NCH // NS)
    LAG = 8

    mesh = plsc.VectorSubcoreMesh(
        core_axis_name="c", subcore_axis_name="s", num_cores=NC, num_subcores=NS
    )

    @functools.partial(
        pl.kernel,
        out_type=jax.ShapeDtypeStruct((NC, N, 128), jnp.float32),
        mesh=mesh,
        scratch_types=[
            pltpu.VMEM((NCHK, CH), jnp.int32),
            pltpu.VMEM((CH, 16), jnp.float32),
            pltpu.VMEM((RCH, 16), jnp.float32),
            pltpu.VMEM_SHARED((NA, 16), jnp.float32),
            pltpu.SemaphoreType.DMA,
        ],
        compiler_params=pltpu.CompilerParams(use_tc_tiling_on_sc=False),
    )
    def degk(dst_hbm, deg_out, dstv, ones_b, small, deg_sh, dsem):
        c = lax.axis_index("c")
        s = lax.axis_index("s")
        w = s * NC + c
        z16 = jnp.zeros((16,), jnp.float32)
        one16 = jnp.ones((16,), jnp.float32)

        pltpu.sync_copy(dst_hbm.at[w], dstv)

        def zsmall_body(i, carry):
            small[i, :] = z16
            return carry

        lax.fori_loop(0, RCH, zsmall_body, 0)

        def ones_body(i, carry):
            ones_b[i, :] = one16
            return carry

        lax.fori_loop(0, CH, ones_body, 0)

        for i in range(CPT):
            t = s + NS * i
            @pl.when(t < NCH)
            def _():
                pltpu.sync_copy(small, deg_sh.at[pl.ds(t * RCH, RCH)])
        plsc.subcore_barrier()

        def blk(k, carry):
            pltpu.async_copy(ones_b, deg_sh.at[dstv.at[k]], dsem, add=True)
            @pl.when(k >= LAG)
            def _():
                pltpu.make_async_copy(
                    ones_b, deg_sh.at[dstv.at[0]], dsem).wait()
            return carry

        lax.fori_loop(0, NCHK, blk, 0)
        for _ in range(LAG):
            pltpu.make_async_copy(ones_b, deg_sh.at[dstv.at[0]], dsem).wait()
        plsc.subcore_barrier()

        for i in range(CPT):
            t = s + NS * i
            @pl.when(t < NCH)
            def _():
                r0 = t * RCH
                pltpu.sync_copy(deg_sh.at[pl.ds(r0, RCH)], small)
                pltpu.sync_copy(
                    small, deg_out.at[c, pl.ds(r0, RCH), pl.ds(0, 16)])

    return degk(dst3)


def kernel(x, edge_index, W1l, b1, W1r, W2l, b2, W2r):
    N = x.shape[0]
    E = edge_index.shape[1]
    ET = E // NW
    ETP = -(-ET // CH) * CH
    src3 = edge_index[0].reshape(NW, ET // CH, CH)
    dst3 = edge_index[1].reshape(NW, ET // CH, CH)

    deg = _sc_degree(dst3, N)
    Pa, Pb, S1 = _tc_proj(x, W1l, W1r, b1)
    acc1 = _sc_segment(Pa, Pb, src3, dst3)
    h = _tc_combine(acc1, deg, S1)

    Qa, Qb, S2 = _tc_proj(h, W2l, W2r, b2)
    acc2 = _sc_segment(Qa, Qb, src3, dst3)
    return _tc_combine(acc2, deg, S2)


# speedup vs baseline: 11.0353x; 1.2970x over previous
import functools

import jax
import jax.numpy as jnp
from jax import lax
from jax.experimental import pallas as pl
from jax.experimental.pallas import tpu as pltpu
from jax.experimental.pallas import tpu_sc as plsc

NC = 2
NS = 16
NW = NC * NS
CH = 125
HH = 64
NB = 4
RCH = 200


def _tc_proj(x, Wl, Wr, b):
    N, D = x.shape
    H = Wl.shape[1]
    R = 1000
    G = N // R

    def body(x_ref, wl_ref, wr_ref, b_ref, p0_ref, p1_ref, s_ref):
        xb = x_ref[...]
        pv = jnp.dot(xb, wl_ref[...], preferred_element_type=jnp.float32)
        p0_ref[...] = pv[:, :HH]
        p1_ref[...] = pv[:, HH:]
        s_ref[...] = jnp.dot(xb, wr_ref[...], preferred_element_type=jnp.float32) + b_ref[...]

    return pl.pallas_call(
        body,
        grid=(G,),
        in_specs=[
            pl.BlockSpec((R, D), lambda i: (i, 0)),
            pl.BlockSpec((D, H), lambda i: (0, 0)),
            pl.BlockSpec((D, H), lambda i: (0, 0)),
            pl.BlockSpec((1, H), lambda i: (0, 0)),
        ],
        out_specs=[
            pl.BlockSpec((R, HH), lambda i: (i, 0)),
            pl.BlockSpec((R, HH), lambda i: (i, 0)),
            pl.BlockSpec((R, H), lambda i: (i, 0)),
        ],
        out_shape=[
            jax.ShapeDtypeStruct((N, HH), jnp.float32),
            jax.ShapeDtypeStruct((N, HH), jnp.float32),
            jax.ShapeDtypeStruct((N, H), jnp.float32),
        ],
    )(x, Wl, Wr, b.reshape(1, H))


def _tc_combine(acc, deg, S):
    N, H = S.shape
    R = 1000
    G = N // R

    def body(a0_ref, a1_ref, d0_ref, d1_ref, s_ref, o_ref):
        d = d0_ref[0, :, 0:1] + d1_ref[0, :, 0:1]
        a = a0_ref[0] + a1_ref[0]
        o_ref[...] = jnp.maximum(a / jnp.maximum(d, 1.0) + s_ref[...], 0.0)

    full = pl.BlockSpec((R, H), lambda i: (i, 0))
    c0 = pl.BlockSpec((1, R, 128), lambda i: (0, i, 0))
    c1 = pl.BlockSpec((1, R, 128), lambda i: (1, i, 0))
    return pl.pallas_call(
        body,
        grid=(G,),
        in_specs=[c0, c1, c0, c1, full],
        out_specs=full,
        out_shape=jax.ShapeDtypeStruct((N, H), jnp.float32),
    )(acc, acc, deg, deg, S)


def _sc_segment(P0, P1, src3, dst3):
    N = P0.shape[0]
    NCHK = dst3.shape[1]
    NR = NCHK // NB
    NA = N + 8
    NCH = N // RCH
    CPT = -(-NCH // NS)
    HL = HH // 16

    mesh = plsc.VectorSubcoreMesh(
        core_axis_name="c", subcore_axis_name="s", num_cores=NC, num_subcores=NS
    )

    @functools.partial(
        pl.kernel,
        out_type=jax.ShapeDtypeStruct((NC, N, 2 * HH), jnp.float32),
        mesh=mesh,
        scratch_types=[
            pltpu.VMEM((NCHK, CH), jnp.int32),
            pltpu.VMEM((NCHK, CH), jnp.int32),
            pltpu.VMEM((NB, CH, HH), jnp.float32),
            pltpu.VMEM((RCH, HH), jnp.float32),
            pltpu.VMEM((RCH, HH), jnp.float32),
            pltpu.VMEM_SHARED((NA, HH), jnp.float32),
            pltpu.SemaphoreType.DMA((NB,)),
            pltpu.SemaphoreType.DMA((NB,)),
        ],
        compiler_params=pltpu.CompilerParams(use_tc_tiling_on_sc=False),
    )
    def seg(p0_hbm, p1_hbm, src_hbm, dst_hbm, acc_out,
            srcv, dstv, ring, zbuf, obuf, acc_sh, gsem, ssem):
        c = lax.axis_index("c")
        s = lax.axis_index("s")
        w = s * NC + c
        z16 = jnp.zeros((16,), jnp.float32)

        pltpu.sync_copy(src_hbm.at[w], srcv)
        pltpu.sync_copy(dst_hbm.at[w], dstv)

        def zbuf_body(i, carry):
            zbuf[i // HL, pl.ds((i % HL) * 16, 16)] = z16
            return carry

        lax.fori_loop(0, RCH * HL, zbuf_body, 0)

        for hc in range(2):
            p_hbm = (p0_hbm, p1_hbm)[hc]
            for i in range(CPT):
                t = s + NS * i
                @pl.when(t < NCH)
                def _():
                    pltpu.sync_copy(zbuf, acc_sh.at[pl.ds(t * RCH, RCH)])
            plsc.subcore_barrier()

            pltpu.async_copy(p_hbm.at[srcv.at[0]], ring.at[0], gsem.at[0])
            pltpu.async_copy(p_hbm.at[srcv.at[1]], ring.at[1], gsem.at[1])

            def rnd(r, carry):
                for b in range(NB):
                    k = r * NB + b
                    b2 = (b + 2) % NB
                    pltpu.make_async_copy(
                        p_hbm.at[srcv.at[k]], ring.at[b], gsem.at[b]).wait()
                    pltpu.async_copy(
                        ring.at[b], acc_sh.at[dstv.at[k]], ssem.at[b], add=True)
                    if b < 2:
                        @pl.when(r >= 1)
                        def _():
                            pltpu.make_async_copy(
                                ring.at[b2], acc_sh.at[dstv.at[k]],
                                ssem.at[b2]).wait()
                            pltpu.async_copy(
                                p_hbm.at[srcv.at[k + 2]], ring.at[b2],
                                gsem.at[b2])
                        @pl.when(r == 0)
                        def _():
                            pltpu.async_copy(
                                p_hbm.at[srcv.at[k + 2]], ring.at[b2],
                                gsem.at[b2])
                    else:
                        pltpu.make_async_copy(
                            ring.at[b2], acc_sh.at[dstv.at[k]],
                            ssem.at[b2]).wait()
                        @pl.when(r < NR - 1)
                        def _():
                            pltpu.async_copy(
                                p_hbm.at[srcv.at[k + 2]], ring.at[b2],
                                gsem.at[b2])
                return carry

            lax.fori_loop(0, NR, rnd, 0)

            for b in range(2, NB):
                pltpu.make_async_copy(
                    ring.at[b], acc_sh.at[dstv.at[b]], ssem.at[b]).wait()
            plsc.subcore_barrier()

            for i in range(CPT):
                t = s + NS * i
                @pl.when(t < NCH)
                def _():
                    r0 = t * RCH
                    pltpu.sync_copy(acc_sh.at[pl.ds(r0, RCH)], obuf)
                    pltpu.sync_copy(
                        obuf, acc_out.at[c, pl.ds(r0, RCH), pl.ds(hc * HH, HH)])

    return seg(P0, P1, src3, dst3)


def _sc_degree(dst3, N):
    NCHK = dst3.shape[1]
    NA = N + 8
    NCH = N // RCH
    CPT = -(-NCH // NS)
    LAG = 8

    mesh = plsc.VectorSubcoreMesh(
        core_axis_name="c", subcore_axis_name="s", num_cores=NC, num_subcores=NS
    )

    @functools.partial(
        pl.kernel,
        out_type=jax.ShapeDtypeStruct((NC, N, 128), jnp.float32),
        mesh=mesh,
        scratch_types=[
            pltpu.VMEM((NCHK, CH), jnp.int32),
            pltpu.VMEM((CH, 16), jnp.float32),
            pltpu.VMEM((RCH, 16), jnp.float32),
            pltpu.VMEM_SHARED((NA, 16), jnp.float32),
            pltpu.SemaphoreType.DMA,
        ],
        compiler_params=pltpu.CompilerParams(use_tc_tiling_on_sc=False),
    )
    def degk(dst_hbm, deg_out, dstv, ones_b, small, deg_sh, dsem):
        c = lax.axis_index("c")
        s = lax.axis_index("s")
        w = s * NC + c
        z16 = jnp.zeros((16,), jnp.float32)
        one16 = jnp.ones((16,), jnp.float32)

        pltpu.sync_copy(dst_hbm.at[w], dstv)

        def zsmall_body(i, carry):
            small[i, :] = z16
            return carry

        lax.fori_loop(0, RCH, zsmall_body, 0)

        def ones_body(i, carry):
            ones_b[i, :] = one16
            return carry

        lax.fori_loop(0, CH, ones_body, 0)

        for i in range(CPT):
            t = s + NS * i
            @pl.when(t < NCH)
            def _():
                pltpu.sync_copy(small, deg_sh.at[pl.ds(t * RCH, RCH)])
        plsc.subcore_barrier()

        def blk(k, carry):
            pltpu.async_copy(ones_b, deg_sh.at[dstv.at[k]], dsem, add=True)
            @pl.when(k >= LAG)
            def _():
                pltpu.make_async_copy(
                    ones_b, deg_sh.at[dstv.at[0]], dsem).wait()
            return carry

        lax.fori_loop(0, NCHK, blk, 0)
        for _ in range(LAG):
            pltpu.make_async_copy(ones_b, deg_sh.at[dstv.at[0]], dsem).wait()
        plsc.subcore_barrier()

        for i in range(CPT):
            t = s + NS * i
            @pl.when(t < NCH)
            def _():
                r0 = t * RCH
                pltpu.sync_copy(deg_sh.at[pl.ds(r0, RCH)], small)
                pltpu.sync_copy(
                    small, deg_out.at[c, pl.ds(r0, RCH), pl.ds(0, 16)])

    return degk(dst3)


def kernel(x, edge_index, W1l, b1, W1r, W2l, b2, W2r):
    N = x.shape[0]
    E = edge_index.shape[1]
    ET = E // NW
    ETP = -(-ET // CH) * CH
    src3 = edge_index[0].reshape(NW, ET // CH, CH)
    dst3 = edge_index[1].reshape(NW, ET // CH, CH)

    deg = _sc_degree(dst3, N)
    Pa, Pb, S1 = _tc_proj(x, W1l, W1r, b1)
    acc1 = _sc_segment(Pa, Pb, src3, dst3)
    h = _tc_combine(acc1, deg, S1)

    Qa, Qb, S2 = _tc_proj(h, W2l, W2r, b2)
    acc2 = _sc_segment(Qa, Qb, src3, dst3)
    return _tc_combine(acc2, deg, S2)


# speedup vs baseline: 12.3439x; 1.1186x over previous
import functools

import jax
import jax.numpy as jnp
from jax import lax
from jax.experimental import pallas as pl
from jax.experimental.pallas import tpu as pltpu
from jax.experimental.pallas import tpu_sc as plsc

NC = 2
NS = 16
NW = NC * NS
CH = 125
HH = 64
NB = 5
RCH = 200


def _tc_proj(x, Wl, Wr, b):
    N, D = x.shape
    H = Wl.shape[1]
    R = 1000
    G = N // R

    def body(x_ref, wl_ref, wr_ref, b_ref, p0_ref, p1_ref, s_ref):
        xb = x_ref[...]
        pv = jnp.dot(xb, wl_ref[...], preferred_element_type=jnp.float32)
        p0_ref[...] = pv[:, :HH]
        p1_ref[...] = pv[:, HH:]
        s_ref[...] = jnp.dot(xb, wr_ref[...], preferred_element_type=jnp.float32) + b_ref[...]

    return pl.pallas_call(
        body,
        grid=(G,),
        in_specs=[
            pl.BlockSpec((R, D), lambda i: (i, 0)),
            pl.BlockSpec((D, H), lambda i: (0, 0)),
            pl.BlockSpec((D, H), lambda i: (0, 0)),
            pl.BlockSpec((1, H), lambda i: (0, 0)),
        ],
        out_specs=[
            pl.BlockSpec((R, HH), lambda i: (i, 0)),
            pl.BlockSpec((R, HH), lambda i: (i, 0)),
            pl.BlockSpec((R, H), lambda i: (i, 0)),
        ],
        out_shape=[
            jax.ShapeDtypeStruct((N, HH), jnp.float32),
            jax.ShapeDtypeStruct((N, HH), jnp.float32),
            jax.ShapeDtypeStruct((N, H), jnp.float32),
        ],
    )(x, Wl, Wr, b.reshape(1, H))


def _tc_combine(acc, deg, S):
    N, H = S.shape
    R = 1000
    G = N // R

    def body(a0_ref, a1_ref, d0_ref, d1_ref, s_ref, o_ref):
        d = d0_ref[0, :, 0:1] + d1_ref[0, :, 0:1]
        a = a0_ref[0] + a1_ref[0]
        o_ref[...] = jnp.maximum(a / jnp.maximum(d, 1.0) + s_ref[...], 0.0)

    full = pl.BlockSpec((R, H), lambda i: (i, 0))
    c0 = pl.BlockSpec((1, R, 128), lambda i: (0, i, 0))
    c1 = pl.BlockSpec((1, R, 128), lambda i: (1, i, 0))
    return pl.pallas_call(
        body,
        grid=(G,),
        in_specs=[c0, c1, c0, c1, full],
        out_specs=full,
        out_shape=jax.ShapeDtypeStruct((N, H), jnp.float32),
    )(acc, acc, deg, deg, S)


def _sc_segment(P0, P1, src3, dst3):
    N = P0.shape[0]
    NCHK = dst3.shape[1]
    NR = NCHK // NB
    NA = N + 8
    NCH = N // RCH
    CPT = -(-NCH // NS)
    HL = HH // 16

    mesh = plsc.VectorSubcoreMesh(
        core_axis_name="c", subcore_axis_name="s", num_cores=NC, num_subcores=NS
    )

    @functools.partial(
        pl.kernel,
        out_type=jax.ShapeDtypeStruct((NC, N, 2 * HH), jnp.float32),
        mesh=mesh,
        scratch_types=[
            pltpu.VMEM((NCHK, CH), jnp.int32),
            pltpu.VMEM((NCHK, CH), jnp.int32),
            pltpu.VMEM((NB, CH, HH), jnp.float32),
            pltpu.VMEM((RCH, HH), jnp.float32),
            pltpu.VMEM((RCH, HH), jnp.float32),
            pltpu.VMEM_SHARED((NA, HH), jnp.float32),
            pltpu.SemaphoreType.DMA((NB,)),
            pltpu.SemaphoreType.DMA((NB,)),
        ],
        compiler_params=pltpu.CompilerParams(use_tc_tiling_on_sc=False),
    )
    def seg(p0_hbm, p1_hbm, src_hbm, dst_hbm, acc_out,
            srcv, dstv, ring, zbuf, obuf, acc_sh, gsem, ssem):
        c = lax.axis_index("c")
        s = lax.axis_index("s")
        w = s * NC + c
        z16 = jnp.zeros((16,), jnp.float32)

        pltpu.sync_copy(src_hbm.at[w], srcv)
        pltpu.sync_copy(dst_hbm.at[w], dstv)

        def zbuf_body(i, carry):
            zbuf[i // HL, pl.ds((i % HL) * 16, 16)] = z16
            return carry

        lax.fori_loop(0, RCH * HL, zbuf_body, 0)

        for hc in range(2):
            p_hbm = (p0_hbm, p1_hbm)[hc]
            for i in range(CPT):
                t = s + NS * i
                @pl.when(t < NCH)
                def _():
                    pltpu.sync_copy(zbuf, acc_sh.at[pl.ds(t * RCH, RCH)])
            plsc.subcore_barrier()

            LK = NB - 2
            for j in range(LK):
                pltpu.async_copy(p_hbm.at[srcv.at[j]], ring.at[j], gsem.at[j])

            def rnd(r, carry):
                for b in range(NB):
                    k = r * NB + b
                    b2 = (b + LK) % NB
                    pltpu.make_async_copy(
                        p_hbm.at[srcv.at[k]], ring.at[b], gsem.at[b]).wait()
                    pltpu.async_copy(
                        ring.at[b], acc_sh.at[dstv.at[k]], ssem.at[b], add=True)
                    @pl.when(k >= 2)
                    def _():
                        pltpu.make_async_copy(
                            ring.at[b2], acc_sh.at[dstv.at[k]],
                            ssem.at[b2]).wait()
                    @pl.when(k + LK < NCHK)
                    def _():
                        pltpu.async_copy(
                            p_hbm.at[srcv.at[k + LK]], ring.at[b2], gsem.at[b2])
                return carry

            lax.fori_loop(0, NR, rnd, 0)

            for j in range(2):
                b = (NCHK - 2 + j) % NB
                pltpu.make_async_copy(
                    ring.at[b], acc_sh.at[dstv.at[0]], ssem.at[b]).wait()
            plsc.subcore_barrier()

            for i in range(CPT):
                t = s + NS * i
                @pl.when(t < NCH)
                def _():
                    r0 = t * RCH
                    pltpu.sync_copy(acc_sh.at[pl.ds(r0, RCH)], obuf)
                    pltpu.sync_copy(
                        obuf, acc_out.at[c, pl.ds(r0, RCH), pl.ds(hc * HH, HH)])

    return seg(P0, P1, src3, dst3)


def _sc_degree(dst3, N):
    NCHK = dst3.shape[1]
    NA = N + 8
    NCH = N // RCH
    CPT = -(-NCH // NS)
    LAG = 8

    mesh = plsc.VectorSubcoreMesh(
        core_axis_name="c", subcore_axis_name="s", num_cores=NC, num_subcores=NS
    )

    @functools.partial(
        pl.kernel,
        out_type=jax.ShapeDtypeStruct((NC, N, 128), jnp.float32),
        mesh=mesh,
        scratch_types=[
            pltpu.VMEM((NCHK, CH), jnp.int32),
            pltpu.VMEM((CH, 16), jnp.float32),
            pltpu.VMEM((RCH, 16), jnp.float32),
            pltpu.VMEM_SHARED((NA, 16), jnp.float32),
            pltpu.SemaphoreType.DMA,
        ],
        compiler_params=pltpu.CompilerParams(use_tc_tiling_on_sc=False),
    )
    def degk(dst_hbm, deg_out, dstv, ones_b, small, deg_sh, dsem):
        c = lax.axis_index("c")
        s = lax.axis_index("s")
        w = s * NC + c
        z16 = jnp.zeros((16,), jnp.float32)
        one16 = jnp.ones((16,), jnp.float32)

        pltpu.sync_copy(dst_hbm.at[w], dstv)

        def zsmall_body(i, carry):
            small[i, :] = z16
            return carry

        lax.fori_loop(0, RCH, zsmall_body, 0)

        def ones_body(i, carry):
            ones_b[i, :] = one16
            return carry

        lax.fori_loop(0, CH, ones_body, 0)

        for i in range(CPT):
            t = s + NS * i
            @pl.when(t < NCH)
            def _():
                pltpu.sync_copy(small, deg_sh.at[pl.ds(t * RCH, RCH)])
        plsc.subcore_barrier()

        def blk(k, carry):
            pltpu.async_copy(ones_b, deg_sh.at[dstv.at[k]], dsem, add=True)
            @pl.when(k >= LAG)
            def _():
                pltpu.make_async_copy(
                    ones_b, deg_sh.at[dstv.at[0]], dsem).wait()
            return carry

        lax.fori_loop(0, NCHK, blk, 0)
        for _ in range(LAG):
            pltpu.make_async_copy(ones_b, deg_sh.at[dstv.at[0]], dsem).wait()
        plsc.subcore_barrier()

        for i in range(CPT):
            t = s + NS * i
            @pl.when(t < NCH)
            def _():
                r0 = t * RCH
                pltpu.sync_copy(deg_sh.at[pl.ds(r0, RCH)], small)
                pltpu.sync_copy(
                    small, deg_out.at[c, pl.ds(r0, RCH), pl.ds(0, 16)])

    return degk(dst3)


def kernel(x, edge_index, W1l, b1, W1r, W2l, b2, W2r):
    N = x.shape[0]
    E = edge_index.shape[1]
    ET = E // NW
    ETP = -(-ET // CH) * CH
    src3 = edge_index[0].reshape(NW, ET // CH, CH)
    dst3 = edge_index[1].reshape(NW, ET // CH, CH)

    deg = _sc_degree(dst3, N)
    Pa, Pb, S1 = _tc_proj(x, W1l, W1r, b1)
    acc1 = _sc_segment(Pa, Pb, src3, dst3)
    h = _tc_combine(acc1, deg, S1)

    Qa, Qb, S2 = _tc_proj(h, W2l, W2r, b2)
    acc2 = _sc_segment(Qa, Qb, src3, dst3)
    return _tc_combine(acc2, deg, S2)
